# Initial kernel scaffold; baseline (speedup 1.0000x reference)
import jax
import jax.numpy as jnp
from jax.experimental import pallas as pl


def kernel(x, weight):
    raise NotImplementedError("write your pallas kernel here")


# speedup vs baseline: 2.9686x; 2.9686x over previous
import functools

import jax
import jax.numpy as jnp
from jax import lax
from jax.experimental import pallas as pl
from jax.experimental.pallas import tpu as pltpu
from jax.experimental.pallas import tpu_sc as plsc

DIM = 128
CHUNK = 128


@functools.partial(jax.jit, static_argnums=())
def _sc_gather(table, idx3d):
    info = plsc.get_sparse_core_info()
    nc, ns = info.num_cores, info.num_subcores
    nw = nc * ns
    assert idx3d.shape[0] == nw
    chunks_per_w = idx3d.shape[1]
    rows_per_w = chunks_per_w * CHUNK
    total = nw * rows_per_w

    mesh = plsc.VectorSubcoreMesh(core_axis_name="c", subcore_axis_name="s")

    @functools.partial(
        pl.kernel,
        mesh=mesh,
        out_type=jax.ShapeDtypeStruct((total, DIM), jnp.float32),
        scratch_types=[
            pltpu.VMEM((chunks_per_w, CHUNK), jnp.int32),
            pltpu.VMEM((CHUNK, DIM), jnp.float32),
            pltpu.SemaphoreType.DMA,
        ],
    )
    def k(table_hbm, idx_hbm, out_hbm, idx_v, rows_v, sem):
        wid = lax.axis_index("s") * nc + lax.axis_index("c")
        obase = wid * rows_per_w
        pltpu.sync_copy(idx_hbm.at[wid], idx_v)

        def body(j, carry):
            pltpu.async_copy(table_hbm.at[idx_v.at[j]], rows_v, sem).wait()
            pltpu.sync_copy(rows_v, out_hbm.at[pl.ds(obase + j * CHUNK, CHUNK)])
            return carry

        lax.fori_loop(0, chunks_per_w, body, 0)

    return k(table, idx3d)


def kernel(x, weight):
    nw = 32
    flat = x.reshape(-1).astype(jnp.int32)
    idx3d = flat.reshape(nw, flat.size // (nw * CHUNK), CHUNK)
    out = _sc_gather(weight, idx3d)
    return out.reshape(x.shape + (DIM,))


# speedup vs baseline: 3.3206x; 1.1186x over previous
import functools

import jax
import jax.numpy as jnp
from jax import lax
from jax.experimental import pallas as pl
from jax.experimental.pallas import tpu as pltpu
from jax.experimental.pallas import tpu_sc as plsc

DIM = 128
CHUNK = 128


@functools.partial(jax.jit, static_argnums=())
def _sc_gather(table, idx3d):
    info = plsc.get_sparse_core_info()
    nc, ns = info.num_cores, info.num_subcores
    nw = nc * ns
    assert idx3d.shape[0] == nw
    chunks_per_w = idx3d.shape[1]
    rows_per_w = chunks_per_w * CHUNK
    total = nw * rows_per_w

    mesh = plsc.VectorSubcoreMesh(core_axis_name="c", subcore_axis_name="s")

    @functools.partial(
        pl.kernel,
        mesh=mesh,
        out_type=jax.ShapeDtypeStruct((total, DIM), jnp.float32),
        scratch_types=[
            pltpu.VMEM((chunks_per_w, CHUNK), jnp.int32),
            pltpu.VMEM((CHUNK, DIM), jnp.float32),
            pltpu.VMEM((CHUNK, DIM), jnp.float32),
            pltpu.SemaphoreType.DMA,
            pltpu.SemaphoreType.DMA,
        ],
    )
    def k(table_hbm, idx_hbm, out_hbm, idx_v, rows0, rows1, sem0, sem1):
        wid = lax.axis_index("s") * nc + lax.axis_index("c")
        obase = wid * rows_per_w
        pltpu.sync_copy(idx_hbm.at[wid], idx_v)

        bufs = ((rows0, sem0), (rows1, sem1))

        def gstart(j, b):
            rows, sem = bufs[b]
            pltpu.async_copy(table_hbm.at[idx_v.at[j]], rows, sem)

        def drain(j, b, lookahead):
            rows, sem = bufs[b]
            pltpu.make_async_copy(table_hbm.at[idx_v.at[j]], rows, sem).wait()
            pltpu.sync_copy(rows, out_hbm.at[pl.ds(obase + j * CHUNK, CHUNK)])
            if lookahead:
                gstart(j + 2, b)

        gstart(0, 0)
        gstart(1, 1)

        def body(g, carry):
            j = g * 2
            drain(j, 0, True)
            drain(j + 1, 1, True)
            return carry

        lax.fori_loop(0, (chunks_per_w - 2) // 2, body, 0)
        drain(chunks_per_w - 2, 0, False)
        drain(chunks_per_w - 1, 1, False)

    return k(table, idx3d)


def kernel(x, weight):
    nw = 32
    flat = x.reshape(-1).astype(jnp.int32)
    idx3d = flat.reshape(nw, flat.size // (nw * CHUNK), CHUNK)
    out = _sc_gather(weight, idx3d)
    return out.reshape(x.shape + (DIM,))


# speedup vs baseline: 3.3296x; 1.0027x over previous
import functools

import jax
import jax.numpy as jnp
from jax import lax
from jax.experimental import pallas as pl
from jax.experimental.pallas import tpu as pltpu
from jax.experimental.pallas import tpu_sc as plsc

DIM = 128
CHUNK = 128
NBUF = 4


def _sc_gather(table, idx3d):
    info = plsc.get_sparse_core_info()
    nc, ns = info.num_cores, info.num_subcores
    nw = nc * ns
    assert idx3d.shape[0] == nw
    chunks_per_w = idx3d.shape[1]
    rows_per_w = chunks_per_w * CHUNK
    total = nw * rows_per_w

    mesh = plsc.VectorSubcoreMesh(core_axis_name="c", subcore_axis_name="s")

    @functools.partial(
        pl.kernel,
        mesh=mesh,
        out_type=jax.ShapeDtypeStruct((total, DIM), jnp.float32),
        scratch_types=[
            pltpu.VMEM((chunks_per_w, CHUNK), jnp.int32),
        ]
        + [pltpu.VMEM((CHUNK, DIM), jnp.float32)] * NBUF
        + [pltpu.SemaphoreType.DMA] * (2 * NBUF),
    )
    def k(table_hbm, idx_hbm, out_hbm, idx_v, *scratch):
        rows = scratch[:NBUF]
        gsem = scratch[NBUF : 2 * NBUF]
        osem = scratch[2 * NBUF :]
        wid = lax.axis_index("s") * nc + lax.axis_index("c")
        obase = wid * rows_per_w
        pltpu.sync_copy(idx_hbm.at[wid], idx_v)

        def gstart(j, b):
            pltpu.async_copy(table_hbm.at[idx_v.at[j]], rows[b], gsem[b])

        def step(j, b, do_owait, do_gstart):
            pltpu.make_async_copy(table_hbm.at[idx_v.at[j]], rows[b], gsem[b]).wait()
            pltpu.async_copy(
                rows[b], out_hbm.at[pl.ds(obase + j * CHUNK, CHUNK)], osem[b]
            )
            if do_gstart:
                jn = j + 2
                bn = (b + 2) % NBUF
                if do_owait:
                    pltpu.make_async_copy(
                        rows[bn], out_hbm.at[pl.ds(obase, CHUNK)], osem[bn]
                    ).wait()
                pltpu.async_copy(table_hbm.at[idx_v.at[jn]], rows[bn], gsem[bn])

        n = chunks_per_w
        gstart(0, 0)
        gstart(1, 1)
        step(0, 0, False, True)
        step(1, 1, False, True)

        def body(g, carry):
            j0 = 2 + g * 4
            for t in range(4):
                step(j0 + t, (2 + t) % NBUF, True, True)
            return carry

        lax.fori_loop(0, (n - 6) // 4, body, 0)
        step(n - 4, (n - 4) % NBUF, True, True)
        step(n - 3, (n - 3) % NBUF, True, True)
        step(n - 2, (n - 2) % NBUF, False, False)
        step(n - 1, (n - 1) % NBUF, False, False)
        for j in range(n - NBUF, n):
            b = j % NBUF
            pltpu.make_async_copy(
                rows[b], out_hbm.at[pl.ds(obase, CHUNK)], osem[b]
            ).wait()

    return k(table, idx3d)


def kernel(x, weight):
    nw = 32
    flat = x.reshape(-1).astype(jnp.int32)
    idx3d = flat.reshape(nw, flat.size // (nw * CHUNK), CHUNK)
    out = _sc_gather(weight, idx3d)
    return out.reshape(x.shape + (DIM,))


# speedup vs baseline: 10.4697x; 3.1444x over previous
import functools

import jax
import jax.numpy as jnp
from jax import lax
from jax.experimental import pallas as pl
from jax.experimental.pallas import tpu as pltpu
from jax.experimental.pallas import tpu_sc as plsc

DIM = 128
CHUNK = 128
NBUF = 4


def _sc_gather(table, idx3d):
    info = plsc.get_sparse_core_info()
    nc, ns = info.num_cores, info.num_subcores
    nw = nc * ns
    chunks_per_w = idx3d.shape[0] // (nw * CHUNK)
    rows_per_w = chunks_per_w * CHUNK
    total = nw * rows_per_w

    mesh = plsc.VectorSubcoreMesh(core_axis_name="c", subcore_axis_name="s")

    @functools.partial(
        pl.kernel,
        mesh=mesh,
        out_type=jax.ShapeDtypeStruct((total, DIM), jnp.float32),
        scratch_types=[
            pltpu.VMEM((chunks_per_w * CHUNK,), jnp.int32),
        ]
        + [pltpu.VMEM((CHUNK, DIM), jnp.float32)] * NBUF
        + [pltpu.SemaphoreType.DMA] * (2 * NBUF),
    )
    def k(table_hbm, idx_hbm, out_hbm, idx_v, *scratch):
        rows = scratch[:NBUF]
        gsem = scratch[NBUF : 2 * NBUF]
        osem = scratch[2 * NBUF :]
        wid = lax.axis_index("s") * nc + lax.axis_index("c")
        obase = wid * rows_per_w
        pltpu.sync_copy(idx_hbm.at[pl.ds(wid * rows_per_w, rows_per_w)], idx_v)

        def gstart(j, b):
            pltpu.async_copy(
                table_hbm.at[idx_v.at[pl.ds(j * CHUNK, CHUNK)]], rows[b], gsem[b]
            )

        def step(j, b, do_owait, do_gstart):
            pltpu.make_async_copy(
                table_hbm.at[idx_v.at[pl.ds(j * CHUNK, CHUNK)]], rows[b], gsem[b]
            ).wait()
            pltpu.async_copy(
                rows[b], out_hbm.at[pl.ds(obase + j * CHUNK, CHUNK)], osem[b]
            )
            if do_gstart:
                jn = j + 2
                bn = (b + 2) % NBUF
                if do_owait:
                    pltpu.make_async_copy(
                        rows[bn], out_hbm.at[pl.ds(obase, CHUNK)], osem[bn]
                    ).wait()
                pltpu.async_copy(
                    table_hbm.at[idx_v.at[pl.ds(jn * CHUNK, CHUNK)]], rows[bn], gsem[bn]
                )

        n = chunks_per_w
        gstart(0, 0)
        gstart(1, 1)
        step(0, 0, False, True)
        step(1, 1, False, True)

        def body(g, carry):
            j0 = 2 + g * 4
            for t in range(4):
                step(j0 + t, (2 + t) % NBUF, True, True)
            return carry

        lax.fori_loop(0, (n - 6) // 4, body, 0)
        step(n - 4, (n - 4) % NBUF, True, True)
        step(n - 3, (n - 3) % NBUF, True, True)
        step(n - 2, (n - 2) % NBUF, False, False)
        step(n - 1, (n - 1) % NBUF, False, False)
        for j in range(n - NBUF, n):
            b = j % NBUF
            pltpu.make_async_copy(
                rows[b], out_hbm.at[pl.ds(obase, CHUNK)], osem[b]
            ).wait()

    return k(table, idx3d)


def kernel(x, weight):
    flat = x.T.reshape(-1).astype(jnp.int32)
    out = _sc_gather(weight, flat)
    return out.reshape(x.shape[1], x.shape[0], DIM).transpose(1, 0, 2)


# speedup vs baseline: 10.5244x; 1.0052x over previous
import functools

import jax
import jax.numpy as jnp
from jax import lax
from jax.experimental import pallas as pl
from jax.experimental.pallas import tpu as pltpu
from jax.experimental.pallas import tpu_sc as plsc

DIM = 128
CHUNK = 128
NBUF = 6
LA = 3


def _sc_gather(table, idx3d):
    info = plsc.get_sparse_core_info()
    nc, ns = info.num_cores, info.num_subcores
    nw = nc * ns
    chunks_per_w = idx3d.shape[0] // (nw * CHUNK)
    rows_per_w = chunks_per_w * CHUNK
    total = nw * rows_per_w

    mesh = plsc.VectorSubcoreMesh(core_axis_name="c", subcore_axis_name="s")

    @functools.partial(
        pl.kernel,
        mesh=mesh,
        out_type=jax.ShapeDtypeStruct((total, DIM), jnp.float32),
        scratch_types=[
            pltpu.VMEM((chunks_per_w * CHUNK,), jnp.int32),
        ]
        + [pltpu.VMEM((CHUNK, DIM), jnp.float32)] * NBUF
        + [pltpu.SemaphoreType.DMA] * (2 * NBUF),
    )
    def k(table_hbm, idx_hbm, out_hbm, idx_v, *scratch):
        rows = scratch[:NBUF]
        gsem = scratch[NBUF : 2 * NBUF]
        osem = scratch[2 * NBUF :]
        wid = lax.axis_index("s") * nc + lax.axis_index("c")
        obase = wid * rows_per_w
        pltpu.sync_copy(idx_hbm.at[pl.ds(wid * rows_per_w, rows_per_w)], idx_v)

        def gstart(j, b):
            pltpu.async_copy(
                table_hbm.at[idx_v.at[pl.ds(j * CHUNK, CHUNK)]], rows[b], gsem[b]
            )

        def step(j, b, do_owait, do_gstart):
            pltpu.make_async_copy(
                table_hbm.at[idx_v.at[pl.ds(j * CHUNK, CHUNK)]], rows[b], gsem[b]
            ).wait()
            pltpu.async_copy(
                rows[b], out_hbm.at[pl.ds(obase + j * CHUNK, CHUNK)], osem[b]
            )
            if do_gstart:
                jn = j + LA
                bn = (b + LA) % NBUF
                if do_owait:
                    pltpu.make_async_copy(
                        rows[bn], out_hbm.at[pl.ds(obase, CHUNK)], osem[bn]
                    ).wait()
                pltpu.async_copy(
                    table_hbm.at[idx_v.at[pl.ds(jn * CHUNK, CHUNK)]], rows[bn], gsem[bn]
                )

        n = chunks_per_w
        for j in range(LA):
            gstart(j, j % NBUF)
        for j in range(NBUF - LA):
            step(j, j % NBUF, False, True)
        nmain = n - NBUF
        nblocks = nmain // NBUF

        def body(g, carry):
            j0 = (NBUF - LA) + g * NBUF
            for t in range(NBUF):
                step(j0 + t, (NBUF - LA + t) % NBUF, True, True)
            return carry

        lax.fori_loop(0, nblocks, body, 0)
        for j in range(NBUF - LA + nblocks * NBUF, n - LA):
            step(j, j % NBUF, True, True)
        for j in range(n - LA, n):
            step(j, j % NBUF, False, False)
        for j in range(n - NBUF, n):
            b = j % NBUF
            pltpu.make_async_copy(
                rows[b], out_hbm.at[pl.ds(obase, CHUNK)], osem[b]
            ).wait()

    return k(table, idx3d)


def kernel(x, weight):
    flat = x.T.reshape(-1).astype(jnp.int32)
    out = _sc_gather(weight, flat)
    return out.reshape(x.shape[1], x.shape[0], DIM).transpose(1, 0, 2)


# speedup vs baseline: 10.5668x; 1.0040x over previous
import functools

import jax
import jax.numpy as jnp
from jax import lax
from jax.experimental import pallas as pl
from jax.experimental.pallas import tpu as pltpu
from jax.experimental.pallas import tpu_sc as plsc

DIM = 128
CHUNK = 128
NBUF = 7
LA = 4


def _sc_gather(table, idx3d):
    info = plsc.get_sparse_core_info()
    nc, ns = info.num_cores, info.num_subcores
    nw = nc * ns
    chunks_per_w = idx3d.shape[0] // (nw * CHUNK)
    rows_per_w = chunks_per_w * CHUNK
    total = nw * rows_per_w

    mesh = plsc.VectorSubcoreMesh(core_axis_name="c", subcore_axis_name="s")

    @functools.partial(
        pl.kernel,
        mesh=mesh,
        out_type=jax.ShapeDtypeStruct((total, DIM), jnp.float32),
        scratch_types=[
            pltpu.VMEM((chunks_per_w * CHUNK,), jnp.int32),
        ]
        + [pltpu.VMEM((CHUNK, DIM), jnp.float32)] * NBUF
        + [pltpu.SemaphoreType.DMA] * (2 * NBUF),
    )
    def k(table_hbm, idx_hbm, out_hbm, idx_v, *scratch):
        rows = scratch[:NBUF]
        gsem = scratch[NBUF : 2 * NBUF]
        osem = scratch[2 * NBUF :]
        wid = lax.axis_index("s") * nc + lax.axis_index("c")
        obase = wid * rows_per_w
        pltpu.sync_copy(idx_hbm.at[pl.ds(wid * rows_per_w, rows_per_w)], idx_v)

        def gstart(j, b):
            pltpu.async_copy(
                table_hbm.at[idx_v.at[pl.ds(j * CHUNK, CHUNK)]], rows[b], gsem[b]
            )

        def step(j, b, do_owait, do_gstart):
            pltpu.make_async_copy(
                table_hbm.at[idx_v.at[pl.ds(j * CHUNK, CHUNK)]], rows[b], gsem[b]
            ).wait()
            pltpu.async_copy(
                rows[b], out_hbm.at[pl.ds(obase + j * CHUNK, CHUNK)], osem[b]
            )
            if do_gstart:
                jn = j + LA
                bn = (b + LA) % NBUF
                if do_owait:
                    pltpu.make_async_copy(
                        rows[bn], out_hbm.at[pl.ds(obase, CHUNK)], osem[bn]
                    ).wait()
                pltpu.async_copy(
                    table_hbm.at[idx_v.at[pl.ds(jn * CHUNK, CHUNK)]], rows[bn], gsem[bn]
                )

        n = chunks_per_w
        for j in range(LA):
            gstart(j, j % NBUF)
        for j in range(NBUF - LA):
            step(j, j % NBUF, False, True)
        nmain = n - NBUF
        nblocks = nmain // NBUF

        def body(g, carry):
            j0 = (NBUF - LA) + g * NBUF
            for t in range(NBUF):
                step(j0 + t, (NBUF - LA + t) % NBUF, True, True)
            return carry

        lax.fori_loop(0, nblocks, body, 0)
        for j in range(NBUF - LA + nblocks * NBUF, n - LA):
            step(j, j % NBUF, True, True)
        for j in range(n - LA, n):
            step(j, j % NBUF, False, False)
        for j in range(n - NBUF, n):
            b = j % NBUF
            pltpu.make_async_copy(
                rows[b], out_hbm.at[pl.ds(obase, CHUNK)], osem[b]
            ).wait()

    return k(table, idx3d)


def kernel(x, weight):
    flat = x.T.reshape(-1).astype(jnp.int32)
    out = _sc_gather(weight, flat)
    return out.reshape(x.shape[1], x.shape[0], DIM).transpose(1, 0, 2)
